# Initial kernel scaffold; baseline (speedup 1.0000x reference)
import jax
import jax.numpy as jnp
from jax.experimental import pallas as pl


def kernel(user, item, user_table, item_table, W1, b1, W2, b2, W3, b3):
    raise NotImplementedError("write your pallas kernel here")


# speedup vs baseline: 6.7147x; 6.7147x over previous
import jax
import jax.numpy as jnp
from jax.experimental import pallas as pl
from jax.experimental.pallas import tpu as pltpu
from jax.experimental.pallas import tpu_sc as plsc

BATCH = 16384
EMB = 128
GATHER_WINDOW = 128


def _sc_gather(user, item, user_table, item_table):
    mesh = plsc.VectorSubcoreMesh(core_axis_name="core", subcore_axis_name="subcore")
    out_type = (
        jax.ShapeDtypeStruct((BATCH, EMB), jnp.float32),
        jax.ShapeDtypeStruct((BATCH, EMB), jnp.float32),
    )

    @pl.kernel(out_type=out_type, mesh=mesh)
    def gather_kernel(u_hbm, i_hbm, ut_hbm, it_hbm, uo_hbm, io_hbm):
        def body(ui_vmem, ii_vmem, uo_vmem, io_vmem):
            pltpu.sync_copy(ut_hbm.at[ui_vmem.at[0]], uo_vmem)
            pltpu.sync_copy(it_hbm.at[ii_vmem.at[0]], io_vmem)

        pltpu.emit_pipeline(
            body,
            grid=(BATCH // GATHER_WINDOW,),
            in_specs=[
                pl.BlockSpec((1, GATHER_WINDOW), lambda i: (0, i)),
                pl.BlockSpec((1, GATHER_WINDOW), lambda i: (0, i)),
            ],
            out_specs=[
                pl.BlockSpec((GATHER_WINDOW, EMB), lambda i: (i, 0)),
                pl.BlockSpec((GATHER_WINDOW, EMB), lambda i: (i, 0)),
            ],
            core_axis_name=("core", "subcore"),
            dimension_semantics=(pltpu.PARALLEL,),
        )(u_hbm, i_hbm, uo_hbm, io_hbm)

    return gather_kernel(
        user.reshape(1, BATCH), item.reshape(1, BATCH), user_table, item_table
    )


def _mlp_body(ue_ref, ie_ref, w1u_ref, w1i_ref, b1_ref, w2t_ref, b2_ref,
              w3_ref, b3_ref, o_ref):
    h = jnp.dot(ue_ref[...], w1u_ref[...], preferred_element_type=jnp.float32)
    h = h + jnp.dot(ie_ref[...], w1i_ref[...], preferred_element_type=jnp.float32)
    h = jnp.maximum(h + b1_ref[...], 0.0)
    h2 = jnp.dot(h, w2t_ref[...], preferred_element_type=jnp.float32)
    h2 = jnp.maximum(h2 + b2_ref[...], 0.0)
    o_ref[...] = jnp.sum(h2 * w3_ref[...], axis=1, keepdims=True) + b3_ref[...]


def _tc_mlp(ue, ie, w1u, w1i, b1, w2t, b2, w3, b3):
    blk = 2048
    grid = (BATCH // blk,)
    hid = EMB // 2
    return pl.pallas_call(
        _mlp_body,
        grid=grid,
        in_specs=[
            pl.BlockSpec((blk, EMB), lambda i: (i, 0)),
            pl.BlockSpec((blk, EMB), lambda i: (i, 0)),
            pl.BlockSpec((EMB, EMB), lambda i: (0, 0)),
            pl.BlockSpec((EMB, EMB), lambda i: (0, 0)),
            pl.BlockSpec((1, EMB), lambda i: (0, 0)),
            pl.BlockSpec((EMB, hid), lambda i: (0, 0)),
            pl.BlockSpec((1, hid), lambda i: (0, 0)),
            pl.BlockSpec((1, hid), lambda i: (0, 0)),
            pl.BlockSpec((1, 1), lambda i: (0, 0)),
        ],
        out_specs=pl.BlockSpec((blk, 1), lambda i: (i, 0)),
        out_shape=jax.ShapeDtypeStruct((BATCH, 1), jnp.float32),
    )(ue, ie, w1u, w1i, b1, w2t, b2, w3, b3)


def kernel(user, item, user_table, item_table, W1, b1, W2, b2, W3, b3):
    ue, ie = _sc_gather(user.astype(jnp.int32), item.astype(jnp.int32),
                        user_table, item_table)
    out = _tc_mlp(
        ue, ie,
        W1[:, :EMB].T, W1[:, EMB:].T, b1.reshape(1, EMB),
        W2.T, b2.reshape(1, EMB // 2),
        W3.reshape(1, EMB // 2), b3.reshape(1, 1),
    )
    return out[:, 0]


# speedup vs baseline: 7.0404x; 1.0485x over previous
import jax
import jax.numpy as jnp
from jax.experimental import pallas as pl
from jax.experimental.pallas import tpu as pltpu
from jax.experimental.pallas import tpu_sc as plsc

BATCH = 16384
EMB = 128
GATHER_WINDOW = 128


def _sc_gather(user, item, user_table, item_table):
    mesh = plsc.VectorSubcoreMesh(core_axis_name="core", subcore_axis_name="subcore")
    out_type = (
        jax.ShapeDtypeStruct((BATCH, EMB), jnp.float32),
        jax.ShapeDtypeStruct((BATCH, EMB), jnp.float32),
    )

    @pl.kernel(out_type=out_type, mesh=mesh,
               scratch_types=[pltpu.SemaphoreType.DMA, pltpu.SemaphoreType.DMA])
    def gather_kernel(u_hbm, i_hbm, ut_hbm, it_hbm, uo_hbm, io_hbm, usem, isem):
        def body(ui_vmem, ii_vmem, uo_vmem, io_vmem):
            cu = pltpu.async_copy(ut_hbm.at[ui_vmem.at[0]], uo_vmem, usem)
            ci = pltpu.async_copy(it_hbm.at[ii_vmem.at[0]], io_vmem, isem)
            cu.wait()
            ci.wait()

        pltpu.emit_pipeline(
            body,
            grid=(BATCH // GATHER_WINDOW,),
            in_specs=[
                pl.BlockSpec((1, GATHER_WINDOW), lambda i: (0, i)),
                pl.BlockSpec((1, GATHER_WINDOW), lambda i: (0, i)),
            ],
            out_specs=[
                pl.BlockSpec((GATHER_WINDOW, EMB), lambda i: (i, 0)),
                pl.BlockSpec((GATHER_WINDOW, EMB), lambda i: (i, 0)),
            ],
            core_axis_name=("core", "subcore"),
            dimension_semantics=(pltpu.PARALLEL,),
        )(u_hbm, i_hbm, uo_hbm, io_hbm)

    return gather_kernel(
        user.reshape(1, BATCH), item.reshape(1, BATCH), user_table, item_table
    )


def _mlp_body(ue_ref, ie_ref, w1u_ref, w1i_ref, b1_ref, w2t_ref, b2_ref,
              w3_ref, b3_ref, o_ref):
    h = jnp.dot(ue_ref[...], w1u_ref[...], preferred_element_type=jnp.float32)
    h = h + jnp.dot(ie_ref[...], w1i_ref[...], preferred_element_type=jnp.float32)
    h = jnp.maximum(h + b1_ref[...], 0.0)
    h2 = jnp.dot(h, w2t_ref[...], preferred_element_type=jnp.float32)
    h2 = jnp.maximum(h2 + b2_ref[...], 0.0)
    o_ref[...] = jnp.sum(h2 * w3_ref[...], axis=1, keepdims=True) + b3_ref[...]


def _tc_mlp(ue, ie, w1u, w1i, b1, w2t, b2, w3, b3):
    blk = 2048
    grid = (BATCH // blk,)
    hid = EMB // 2
    return pl.pallas_call(
        _mlp_body,
        grid=grid,
        in_specs=[
            pl.BlockSpec((blk, EMB), lambda i: (i, 0)),
            pl.BlockSpec((blk, EMB), lambda i: (i, 0)),
            pl.BlockSpec((EMB, EMB), lambda i: (0, 0)),
            pl.BlockSpec((EMB, EMB), lambda i: (0, 0)),
            pl.BlockSpec((1, EMB), lambda i: (0, 0)),
            pl.BlockSpec((EMB, hid), lambda i: (0, 0)),
            pl.BlockSpec((1, hid), lambda i: (0, 0)),
            pl.BlockSpec((1, hid), lambda i: (0, 0)),
            pl.BlockSpec((1, 1), lambda i: (0, 0)),
        ],
        out_specs=pl.BlockSpec((blk, 1), lambda i: (i, 0)),
        out_shape=jax.ShapeDtypeStruct((BATCH, 1), jnp.float32),
    )(ue, ie, w1u, w1i, b1, w2t, b2, w3, b3)


def kernel(user, item, user_table, item_table, W1, b1, W2, b2, W3, b3):
    ue, ie = _sc_gather(user.astype(jnp.int32), item.astype(jnp.int32),
                        user_table, item_table)
    out = _tc_mlp(
        ue, ie,
        W1[:, :EMB].T, W1[:, EMB:].T, b1.reshape(1, EMB),
        W2.T, b2.reshape(1, EMB // 2),
        W3.reshape(1, EMB // 2), b3.reshape(1, 1),
    )
    return out[:, 0]


# speedup vs baseline: 8.1771x; 1.1615x over previous
import jax
import jax.numpy as jnp
from jax.experimental import pallas as pl
from jax.experimental.pallas import tpu as pltpu
from jax.experimental.pallas import tpu_sc as plsc

BATCH = 16384
EMB = 128
HID = EMB // 2
GATHER_WINDOW = 128
BLK = 2048
ROWTILES = BLK // 128


def _sc_gather(user, item, user_table, item_table):
    mesh = plsc.VectorSubcoreMesh(core_axis_name="core", subcore_axis_name="subcore")
    out_type = (
        jax.ShapeDtypeStruct((BATCH, EMB), jnp.float32),
        jax.ShapeDtypeStruct((BATCH, EMB), jnp.float32),
    )

    @pl.kernel(out_type=out_type, mesh=mesh,
               scratch_types=[pltpu.SemaphoreType.DMA, pltpu.SemaphoreType.DMA])
    def gather_kernel(u_hbm, i_hbm, ut_hbm, it_hbm, uo_hbm, io_hbm, usem, isem):
        def body(ui_vmem, ii_vmem, uo_vmem, io_vmem):
            cu = pltpu.async_copy(ut_hbm.at[ui_vmem.at[0]], uo_vmem, usem)
            ci = pltpu.async_copy(it_hbm.at[ii_vmem.at[0]], io_vmem, isem)
            cu.wait()
            ci.wait()

        pltpu.emit_pipeline(
            body,
            grid=(BATCH // GATHER_WINDOW,),
            in_specs=[
                pl.BlockSpec((1, GATHER_WINDOW), lambda i: (0, i)),
                pl.BlockSpec((1, GATHER_WINDOW), lambda i: (0, i)),
            ],
            out_specs=[
                pl.BlockSpec((GATHER_WINDOW, EMB), lambda i: (i, 0)),
                pl.BlockSpec((GATHER_WINDOW, EMB), lambda i: (i, 0)),
            ],
            core_axis_name=("core", "subcore"),
            dimension_semantics=(pltpu.PARALLEL,),
        )(u_hbm, i_hbm, uo_hbm, io_hbm)

    return gather_kernel(
        user.reshape(1, BATCH), item.reshape(1, BATCH), user_table, item_table
    )


def _dot_t(x, w):
    return jax.lax.dot_general(x, w, (((1,), (1,)), ((), ())),
                               preferred_element_type=jnp.float32)


def _mlp_body(ue_ref, ie_ref, w1_ref, b1_ref, w2_ref, b2_ref,
              w3_ref, b3_ref, o_ref):
    h = _dot_t(ue_ref[...], w1_ref[:, :EMB])
    h = h + _dot_t(ie_ref[...], w1_ref[:, EMB:])
    h = jnp.maximum(h + b1_ref[...], 0.0)
    h2 = jnp.maximum(_dot_t(h, w2_ref[...]) + b2_ref[...], 0.0)
    res = jnp.sum(h2 * w3_ref[...], axis=1) + b3_ref[0, 0]
    o_ref[...] = res.reshape(ROWTILES, 128)


def _tc_mlp(ue, ie, W1, b1, W2, b2, w3, b3):
    grid = (BATCH // BLK,)
    out = pl.pallas_call(
        _mlp_body,
        grid=grid,
        in_specs=[
            pl.BlockSpec((BLK, EMB), lambda i: (i, 0)),
            pl.BlockSpec((BLK, EMB), lambda i: (i, 0)),
            pl.BlockSpec((EMB, 2 * EMB), lambda i: (0, 0)),
            pl.BlockSpec((1, EMB), lambda i: (0, 0)),
            pl.BlockSpec((HID, EMB), lambda i: (0, 0)),
            pl.BlockSpec((1, HID), lambda i: (0, 0)),
            pl.BlockSpec((1, HID), lambda i: (0, 0)),
            pl.BlockSpec((1, 1), lambda i: (0, 0)),
        ],
        out_specs=pl.BlockSpec((ROWTILES, 128), lambda i: (i, 0)),
        out_shape=jax.ShapeDtypeStruct((BATCH // 128, 128), jnp.float32),
    )(ue, ie, W1, b1, W2, b2, w3, b3)
    return out.reshape(BATCH)


def kernel(user, item, user_table, item_table, W1, b1, W2, b2, W3, b3):
    ue, ie = _sc_gather(user.astype(jnp.int32), item.astype(jnp.int32),
                        user_table, item_table)
    return _tc_mlp(
        ue, ie,
        W1, b1.reshape(1, EMB),
        W2, b2.reshape(1, HID),
        W3.reshape(1, HID), b3.reshape(1, 1),
    )
